# baseline (speedup 1.0000x reference)
import functools

import jax
import jax.numpy as jnp
from jax import lax
from jax.experimental import pallas as pl
from jax.experimental.pallas import tpu as pltpu
from jax.experimental.pallas import tpu_sc as plsc

D_MODEL = 32
CLIP = 128
NUM_EMB = 2 * CLIP + 1
LQ = 2048
LKV = 2048
E2_ROWS = 4168
NW = 32
RPW = LQ // NW

_mesh = plsc.VectorSubcoreMesh(core_axis_name="c", subcore_axis_name="s")


@functools.partial(
    pl.kernel,
    out_type=jax.ShapeDtypeStruct((LQ, LKV, D_MODEL), jnp.float32),
    mesh=_mesh,
    scratch_types=[pltpu.SemaphoreType.DMA],
)
def _band_sc_kernel(e2x_hbm, out_hbm, sem):
    wid = lax.axis_index("s") * 2 + lax.axis_index("c")
    g0 = wid * RPW

    def issue(r, carry):
        g = g0 + r
        o = LQ - g
        k = lax.rem(o, 8)
        o8 = pl.multiple_of(o - k, 8)
        pltpu.make_async_copy(
            e2x_hbm.at[k, pl.ds(o8, LKV), :],
            out_hbm.at[g],
            sem,
        ).start()
        return carry

    lax.fori_loop(0, RPW, issue, 0)

    def drain(r, carry):
        pltpu.make_async_copy(
            e2x_hbm.at[0, pl.ds(0, LKV), :],
            out_hbm.at[g0],
            sem,
        ).wait()
        return carry

    lax.fori_loop(0, RPW, drain, 0)


def kernel(length_q, length_kv, embedding):
    del length_q, length_kv
    top = jnp.broadcast_to(embedding[0:1, :], (1920, D_MODEL))
    bot = jnp.broadcast_to(
        embedding[NUM_EMB - 1:NUM_EMB, :], (E2_ROWS - 1920 - 256, D_MODEL)
    )
    e2 = jnp.concatenate([top, embedding[0:256, :], bot], axis=0)
    e2x = jnp.stack([e2[k:k + E2_ROWS - 8, :] for k in range(8)])
    return _band_sc_kernel(e2x)


# speedup vs baseline: 60.7744x; 60.7744x over previous
import jax
import jax.numpy as jnp
from jax.experimental import pallas as pl
from jax.experimental.pallas import tpu as pltpu

D_MODEL = 32
CLIP = 128
NUM_EMB = 2 * CLIP + 1
LQ = 2048
LKV = 2048
ROW128 = LKV * D_MODEL // 128
E2F_ROWS = 4096 * D_MODEL // 128
BQ = 16


def _band_kernel(e2f_ref, out_ref):
    base = pl.program_id(0) * BQ
    for r in range(BQ):
        g = base + r
        lane_off = ((-r) % 4) * 32
        q = (LQ - g) // 4
        if lane_off == 0:
            out_ref[r] = e2f_ref[pl.ds(q, ROW128), :]
        else:
            a = e2f_ref[pl.ds(q, ROW128), :]
            b = e2f_ref[pl.ds(q + 1, ROW128), :]
            ra = pltpu.roll(a, 128 - lane_off, axis=1)
            rb = pltpu.roll(b, 128 - lane_off, axis=1)
            lane = jax.lax.broadcasted_iota(jnp.int32, (ROW128, 128), 1)
            out_ref[r] = jnp.where(lane < 128 - lane_off, ra, rb)


def kernel(length_q, length_kv, embedding):
    del length_q, length_kv
    top = jnp.broadcast_to(embedding[0:1, :], (1920, D_MODEL))
    bot = jnp.broadcast_to(embedding[NUM_EMB - 1:NUM_EMB, :], (1920, D_MODEL))
    e2f = jnp.concatenate([top, embedding[0:256, :], bot], axis=0).reshape(
        E2F_ROWS, 128
    )
    out = pl.pallas_call(
        _band_kernel,
        grid=(LQ // BQ,),
        in_specs=[pl.BlockSpec((E2F_ROWS, 128), lambda i: (0, 0))],
        out_specs=pl.BlockSpec((BQ, ROW128, 128), lambda i: (i, 0, 0)),
        out_shape=jax.ShapeDtypeStruct((LQ, ROW128, 128), jnp.float32),
    )(e2f)
    return out.reshape(LQ, LKV, D_MODEL)


# speedup vs baseline: 61.4334x; 1.0108x over previous
import jax
import jax.numpy as jnp
from jax.experimental import pallas as pl
from jax.experimental.pallas import tpu as pltpu

D_MODEL = 32
CLIP = 128
NUM_EMB = 2 * CLIP + 1
LQ = 2048
LKV = 2048
ROW128 = LKV * D_MODEL // 128
E2F_ROWS = 4096 * D_MODEL // 128
BQ = 32


def _band_kernel(e2f_ref, out_ref):
    base = pl.program_id(0) * BQ
    for r in range(BQ):
        g = base + r
        lane_off = ((-r) % 4) * 32
        q = (LQ - g) // 4
        if lane_off == 0:
            out_ref[r] = e2f_ref[pl.ds(q, ROW128), :]
        else:
            a = e2f_ref[pl.ds(q, ROW128), :]
            b = e2f_ref[pl.ds(q + 1, ROW128), :]
            ra = pltpu.roll(a, 128 - lane_off, axis=1)
            rb = pltpu.roll(b, 128 - lane_off, axis=1)
            lane = jax.lax.broadcasted_iota(jnp.int32, (ROW128, 128), 1)
            out_ref[r] = jnp.where(lane < 128 - lane_off, ra, rb)


def kernel(length_q, length_kv, embedding):
    del length_q, length_kv
    top = jnp.broadcast_to(embedding[0:1, :], (1920, D_MODEL))
    bot = jnp.broadcast_to(embedding[NUM_EMB - 1:NUM_EMB, :], (1920, D_MODEL))
    e2f = jnp.concatenate([top, embedding[0:256, :], bot], axis=0).reshape(
        E2F_ROWS, 128
    )
    out = pl.pallas_call(
        _band_kernel,
        grid=(LQ // BQ,),
        in_specs=[pl.BlockSpec((E2F_ROWS, 128), lambda i: (0, 0))],
        out_specs=pl.BlockSpec((BQ, ROW128, 128), lambda i: (i, 0, 0)),
        out_shape=jax.ShapeDtypeStruct((LQ, ROW128, 128), jnp.float32),
    )(e2f)
    return out.reshape(LQ, LKV, D_MODEL)
